# baseline (speedup 1.0000x reference)
import dataclasses
import functools

import jax
import jax.numpy as jnp
from jax import lax
from jax.experimental import pallas as pl
from jax.experimental.pallas import tpu as pltpu
from jax.experimental.pallas import tpu_sc as plsc

_N = 10000
_E = 160000
_DIN = 256
_DOUT = 512
_EPS = 1e-3

_NS = 16
_CH = 128
_NCHUNK = 80
_EPW = _CH * _NCHUNK
_EPAD = _NS * _EPW
_NF = 128
_RPW = 624

_f32 = jnp.float32


def _agg_body(xlo_hbm, xhi_hbm, row_hbm, col_hbm, val_hbm, outlo_hbm,
              outhi_hbm, row0, row1, col0, col1, val0, val1, rsc0, rsc1,
              buf0, buf1, acc, sg0, sg1, sm0, sm1, ss0, ss1):
    cid = lax.axis_index("c")
    sid = lax.axis_index("s")
    kbase = sid * _NCHUNK

    rows = (row0, row1)
    cols = (col0, col1)
    vals = (val0, val1)
    rscs = (rsc0, rsc1)
    bufs = (buf0, buf1)
    gsems = (sg0, sg1)
    msems = (sm0, sm1)
    ssems = (ss0, ss1)

    def meta_issue(k, p):
        pltpu.async_copy(row_hbm.at[kbase + k], rows[p], msems[p])
        pltpu.async_copy(col_hbm.at[kbase + k], cols[p], msems[p])
        pltpu.async_copy(val_hbm.at[kbase + k], vals[p], msems[p])

    def meta_wait(k, p):
        pltpu.make_async_copy(row_hbm.at[kbase + k], rows[p], msems[p]).wait()
        pltpu.make_async_copy(col_hbm.at[kbase + k], cols[p], msems[p]).wait()
        pltpu.make_async_copy(val_hbm.at[kbase + k], vals[p], msems[p]).wait()

    def run(x_hbm):
        def gather_issue(p):
            pltpu.async_copy(x_hbm.at[cols[p].at[0]], bufs[p], gsems[p])

        def gather_wait(p):
            pltpu.make_async_copy(x_hbm.at[cols[p].at[0]], bufs[p],
                                  gsems[p]).wait()

        def scale(p):
            buf = bufs[p]

            @pl.loop(0, _CH)
            def _(e):
                zv = jnp.full((16,), 0, jnp.int32)
                ev = jnp.full((16,), e, jnp.int32)
                v = plsc.load_gather(vals[p], [zv, ev])
                for k in range(_NF // 16):
                    sl = pl.ds(k * 16, 16)
                    buf[e, sl] = buf[e, sl] * v

            for k in range(_CH // 16):
                sl = pl.ds(k * 16, 16)
                rscs[p][0, sl] = rows[p][0, sl]

        def scatter_issue(p):
            pltpu.async_copy(bufs[p], acc.at[rscs[p].at[0]], ssems[p],
                             add=True)

        def scatter_wait(p):
            pltpu.make_async_copy(bufs[p], acc.at[rscs[p].at[0]],
                                  ssems[p]).wait()

        meta_issue(0, 0)
        meta_wait(0, 0)
        gather_issue(0)
        meta_issue(1, 1)

        @pl.loop(0, _NCHUNK, step=2)
        def _(j):
            meta_wait(j + 1, 1)
            gather_issue(1)
            gather_wait(0)
            scale(0)
            scatter_issue(0)

            @pl.when(j + 2 < _NCHUNK)
            def _():
                meta_issue(j + 2, 0)

            gather_wait(1)
            scale(1)
            scatter_issue(1)

            @pl.when(j + 3 < _NCHUNK)
            def _():
                meta_issue(j + 3, 1)

            scatter_wait(0)

            @pl.when(j + 2 < _NCHUNK)
            def _():
                meta_wait(j + 2, 0)
                gather_issue(0)

            scatter_wait(1)

    @pl.loop(0, _CH)
    def _(i):
        for k in range(_NF // 16):
            buf0[i, pl.ds(k * 16, 16)] = jnp.zeros((16,), _f32)

    zbase = pl.multiple_of(sid * _RPW, 8)
    for t in range(_RPW // _CH):
        pltpu.sync_copy(buf0, acc.at[pl.ds(zbase + t * _CH, _CH)])
    _zrem = _RPW % _CH
    pltpu.sync_copy(buf0.at[pl.ds(0, _zrem)],
                    acc.at[pl.ds(zbase + (_RPW // _CH) * _CH, _zrem)])

    @pl.when(sid == _NS - 1)
    def _():
        tail = _N - _NS * _RPW
        pltpu.sync_copy(buf0.at[pl.ds(0, tail)],
                        acc.at[pl.ds(_NS * _RPW, tail)])

    plsc.subcore_barrier()

    @pl.when(cid == 0)
    def _():
        run(xlo_hbm)

    @pl.when(cid == 1)
    def _():
        run(xhi_hbm)

    plsc.subcore_barrier()

    def drain(out_hbm):
        dbase = pl.multiple_of(sid * _RPW, 8)
        pltpu.sync_copy(acc.at[pl.ds(dbase, _RPW)],
                        out_hbm.at[pl.ds(dbase, _RPW)])

        @pl.when(sid == _NS - 1)
        def _():
            tail = _N - _NS * _RPW
            pltpu.sync_copy(acc.at[pl.ds(_NS * _RPW, tail)],
                            out_hbm.at[pl.ds(_NS * _RPW, tail)])

    @pl.when(cid == 0)
    def _():
        drain(outlo_hbm)

    @pl.when(cid == 1)
    def _():
        drain(outhi_hbm)


_sc_params = pltpu.CompilerParams()
if "needs_layout_passes" in pltpu.CompilerParams.__dataclass_fields__:
    _sc_params = dataclasses.replace(_sc_params, needs_layout_passes=False)

_agg = functools.partial(
    pl.kernel,
    out_type=(jax.ShapeDtypeStruct((_N, _NF), _f32),
              jax.ShapeDtypeStruct((_N, _NF), _f32)),
    mesh=plsc.VectorSubcoreMesh(core_axis_name="c", subcore_axis_name="s"),
    compiler_params=_sc_params,
    scratch_types=[
        pltpu.VMEM((1, _CH), jnp.int32),
        pltpu.VMEM((1, _CH), jnp.int32),
        pltpu.VMEM((1, _CH), jnp.int32),
        pltpu.VMEM((1, _CH), jnp.int32),
        pltpu.VMEM((1, _CH), _f32),
        pltpu.VMEM((1, _CH), _f32),
        pltpu.VMEM((1, _CH), jnp.int32),
        pltpu.VMEM((1, _CH), jnp.int32),
        pltpu.VMEM((_CH, _NF), _f32),
        pltpu.VMEM((_CH, _NF), _f32),
        pltpu.VMEM_SHARED((_N, _NF), _f32),
        pltpu.SemaphoreType.DMA,
        pltpu.SemaphoreType.DMA,
        pltpu.SemaphoreType.DMA,
        pltpu.SemaphoreType.DMA,
        pltpu.SemaphoreType.DMA,
        pltpu.SemaphoreType.DMA,
    ],
)(_agg_body)


_NB = 10
_BR = _N // _NB


def _stats_body(hlo_ref, hhi_ref, w_ref, gamma_ref, beta_ref, ab_ref,
                c_acc, cs_acc):
    i = pl.program_id(0)

    @pl.when(i == 0)
    def _():
        c_acc[...] = jnp.zeros_like(c_acc)
        cs_acc[...] = jnp.zeros_like(cs_acc)

    hlo = hlo_ref[...]
    hhi = hhi_ref[...]
    dn = (((0,), (0,)), ((), ()))
    c_acc[0:_NF, 0:_NF] += lax.dot_general(hlo, hlo, dn,
                                           preferred_element_type=_f32)
    c_acc[0:_NF, _NF:_DIN] += lax.dot_general(hlo, hhi, dn,
                                              preferred_element_type=_f32)
    c_acc[_NF:_DIN, 0:_NF] += lax.dot_general(hhi, hlo, dn,
                                              preferred_element_type=_f32)
    c_acc[_NF:_DIN, _NF:_DIN] += lax.dot_general(hhi, hhi, dn,
                                                 preferred_element_type=_f32)
    cs_acc[0:1, 0:_NF] += jnp.sum(hlo, axis=0, keepdims=True)
    cs_acc[0:1, _NF:_DIN] += jnp.sum(hhi, axis=0, keepdims=True)

    @pl.when(i == _NB - 1)
    def _():
        w = w_ref[...]
        inv_n = _f32(1.0 / _N)
        mean = jnp.dot(cs_acc[...], w, preferred_element_type=_f32) * inv_n
        cw = jnp.dot(c_acc[...], w, preferred_element_type=_f32)
        ey2 = jnp.sum(w * cw, axis=0, keepdims=True) * inv_n
        var = ey2 - mean * mean
        rstd = lax.rsqrt(var + _EPS)
        a = gamma_ref[...] * rstd
        b = beta_ref[...] - mean * a
        ab_ref[0:1, :] = a
        ab_ref[1:2, :] = b


def _fin_body(hlo_ref, hhi_ref, x_ref, w_ref, ws_ref, ab_ref, o_ref):
    y = (jnp.dot(hlo_ref[...], w_ref[0:_NF, :], preferred_element_type=_f32)
         + jnp.dot(hhi_ref[...], w_ref[_NF:_DIN, :],
                   preferred_element_type=_f32))
    s = jnp.dot(x_ref[...], ws_ref[...], preferred_element_type=_f32)
    a = ab_ref[0:1, :]
    b = ab_ref[1:2, :]
    o_ref[...] = jnp.maximum(y * a + b + s, 0.0)


def kernel(adj_indices, adj_values, node_embs, GCN_weight, skip_weight,
           bn_gamma, bn_beta):
    x = node_embs.astype(_f32)
    pad = _EPAD - _E
    row3 = jnp.concatenate(
        [adj_indices[0].astype(jnp.int32), jnp.zeros((pad,), jnp.int32)]
    ).reshape(_NS * _NCHUNK, 1, _CH)
    col3 = jnp.concatenate(
        [adj_indices[1].astype(jnp.int32), jnp.zeros((pad,), jnp.int32)]
    ).reshape(_NS * _NCHUNK, 1, _CH)
    val3 = jnp.concatenate(
        [adj_values.astype(_f32), jnp.zeros((pad,), _f32)]
    ).reshape(_NS * _NCHUNK, 1, _CH)
    x_lo = x[:, :_NF]
    x_hi = x[:, _NF:]

    h_lo, h_hi = _agg(x_lo, x_hi, row3, col3, val3)

    ab = pl.pallas_call(
        _stats_body,
        grid=(_NB,),
        in_specs=[
            pl.BlockSpec((_BR, _NF), lambda i: (i, 0)),
            pl.BlockSpec((_BR, _NF), lambda i: (i, 0)),
            pl.BlockSpec((_DIN, _DOUT), lambda i: (0, 0)),
            pl.BlockSpec((1, _DOUT), lambda i: (0, 0)),
            pl.BlockSpec((1, _DOUT), lambda i: (0, 0)),
        ],
        out_specs=pl.BlockSpec((2, _DOUT), lambda i: (0, 0)),
        out_shape=jax.ShapeDtypeStruct((2, _DOUT), _f32),
        scratch_shapes=[
            pltpu.VMEM((_DIN, _DIN), _f32),
            pltpu.VMEM((1, _DIN), _f32),
        ],
    )(h_lo, h_hi, GCN_weight, bn_gamma.reshape(1, _DOUT),
      bn_beta.reshape(1, _DOUT))

    out = pl.pallas_call(
        _fin_body,
        grid=(_NB,),
        in_specs=[
            pl.BlockSpec((_BR, _NF), lambda i: (i, 0)),
            pl.BlockSpec((_BR, _NF), lambda i: (i, 0)),
            pl.BlockSpec((_BR, _DIN), lambda i: (i, 0)),
            pl.BlockSpec((_DIN, _DOUT), lambda i: (0, 0)),
            pl.BlockSpec((_DIN, _DOUT), lambda i: (0, 0)),
            pl.BlockSpec((2, _DOUT), lambda i: (0, 0)),
        ],
        out_specs=pl.BlockSpec((_BR, _DOUT), lambda i: (i, 0)),
        out_shape=jax.ShapeDtypeStruct((_N, _DOUT), _f32),
    )(h_lo, h_hi, x, GCN_weight, skip_weight, ab)

    return out


# speedup vs baseline: 1.0131x; 1.0131x over previous
import dataclasses
import functools

import jax
import jax.numpy as jnp
from jax import lax
from jax.experimental import pallas as pl
from jax.experimental.pallas import tpu as pltpu
from jax.experimental.pallas import tpu_sc as plsc

_N = 10000
_E = 160000
_DIN = 256
_DOUT = 512
_EPS = 1e-3

_NS = 16
_CH = 128
_NCHUNK = 80
_EPW = _CH * _NCHUNK
_EPAD = _NS * _EPW
_NF = 128
_NSPLIT = 2
_RPW = 624

_f32 = jnp.float32


def _agg_body(xlo_hbm, xhi_hbm, row_hbm, col_hbm, val_hbm, outlo_hbm,
              outhi_hbm, row0, row1, col0, col1, val0, val1, rsc0, rsc1,
              buf0, buf1, acc, sg0, sg1, sm0, sm1, ss0, ss1):
    cid = lax.axis_index("c")
    sid = lax.axis_index("s")
    kbase = sid * _NCHUNK

    rows = (row0, row1)
    cols = (col0, col1)
    vals = (val0, val1)
    rscs = (rsc0, rsc1)
    bufs = (buf0, buf1)
    gsems = (sg0, sg1)
    msems = (sm0, sm1)
    ssems = (ss0, ss1)

    def meta_issue(k, p):
        pltpu.async_copy(row_hbm.at[kbase + k], rows[p], msems[p])
        pltpu.async_copy(col_hbm.at[kbase + k], cols[p], msems[p])
        pltpu.async_copy(val_hbm.at[kbase + k], vals[p], msems[p])

    def meta_wait(k, p):
        pltpu.make_async_copy(row_hbm.at[kbase + k], rows[p], msems[p]).wait()
        pltpu.make_async_copy(col_hbm.at[kbase + k], cols[p], msems[p]).wait()
        pltpu.make_async_copy(val_hbm.at[kbase + k], vals[p], msems[p]).wait()

    def run(x_hbm):
        def gather_issue(p):
            for h in range(_NSPLIT):
                sub = _CH // _NSPLIT
                pltpu.async_copy(
                    x_hbm.at[cols[p].at[0, pl.ds(h * sub, sub)]],
                    bufs[p].at[pl.ds(h * sub, sub)], gsems[p])

        def gather_wait(p):
            for h in range(_NSPLIT):
                sub = _CH // _NSPLIT
                pltpu.make_async_copy(
                    x_hbm.at[cols[p].at[0, pl.ds(h * sub, sub)]],
                    bufs[p].at[pl.ds(h * sub, sub)], gsems[p]).wait()

        def scale(p):
            buf = bufs[p]

            @pl.loop(0, _CH)
            def _(e):
                zv = jnp.full((16,), 0, jnp.int32)
                ev = jnp.full((16,), e, jnp.int32)
                v = plsc.load_gather(vals[p], [zv, ev])
                for k in range(_NF // 16):
                    sl = pl.ds(k * 16, 16)
                    buf[e, sl] = buf[e, sl] * v

            for k in range(_CH // 16):
                sl = pl.ds(k * 16, 16)
                rscs[p][0, sl] = rows[p][0, sl]

        def scatter_issue(p):
            pltpu.async_copy(bufs[p], acc.at[rscs[p].at[0]], ssems[p],
                             add=True)

        def scatter_wait(p):
            pltpu.make_async_copy(bufs[p], acc.at[rscs[p].at[0]],
                                  ssems[p]).wait()

        meta_issue(0, 0)
        meta_wait(0, 0)
        gather_issue(0)
        meta_issue(1, 1)

        @pl.loop(0, _NCHUNK, step=2)
        def _(j):
            meta_wait(j + 1, 1)
            gather_issue(1)
            gather_wait(0)
            scale(0)
            scatter_issue(0)

            @pl.when(j + 2 < _NCHUNK)
            def _():
                meta_issue(j + 2, 0)

            gather_wait(1)
            scale(1)
            scatter_issue(1)

            @pl.when(j + 3 < _NCHUNK)
            def _():
                meta_issue(j + 3, 1)

            scatter_wait(0)

            @pl.when(j + 2 < _NCHUNK)
            def _():
                meta_wait(j + 2, 0)
                gather_issue(0)

            scatter_wait(1)

    @pl.loop(0, _CH)
    def _(i):
        for k in range(_NF // 16):
            buf0[i, pl.ds(k * 16, 16)] = jnp.zeros((16,), _f32)

    zbase = pl.multiple_of(sid * _RPW, 8)
    for t in range(_RPW // _CH):
        pltpu.sync_copy(buf0, acc.at[pl.ds(zbase + t * _CH, _CH)])
    _zrem = _RPW % _CH
    pltpu.sync_copy(buf0.at[pl.ds(0, _zrem)],
                    acc.at[pl.ds(zbase + (_RPW // _CH) * _CH, _zrem)])

    @pl.when(sid == _NS - 1)
    def _():
        tail = _N - _NS * _RPW
        pltpu.sync_copy(buf0.at[pl.ds(0, tail)],
                        acc.at[pl.ds(_NS * _RPW, tail)])

    plsc.subcore_barrier()

    @pl.when(cid == 0)
    def _():
        run(xlo_hbm)

    @pl.when(cid == 1)
    def _():
        run(xhi_hbm)

    plsc.subcore_barrier()

    def drain(out_hbm):
        dbase = pl.multiple_of(sid * _RPW, 8)
        pltpu.sync_copy(acc.at[pl.ds(dbase, _RPW)],
                        out_hbm.at[pl.ds(dbase, _RPW)])

        @pl.when(sid == _NS - 1)
        def _():
            tail = _N - _NS * _RPW
            pltpu.sync_copy(acc.at[pl.ds(_NS * _RPW, tail)],
                            out_hbm.at[pl.ds(_NS * _RPW, tail)])

    @pl.when(cid == 0)
    def _():
        drain(outlo_hbm)

    @pl.when(cid == 1)
    def _():
        drain(outhi_hbm)


_sc_params = pltpu.CompilerParams()
if "needs_layout_passes" in pltpu.CompilerParams.__dataclass_fields__:
    _sc_params = dataclasses.replace(_sc_params, needs_layout_passes=False)

_agg = functools.partial(
    pl.kernel,
    out_type=(jax.ShapeDtypeStruct((_N, _NF), _f32),
              jax.ShapeDtypeStruct((_N, _NF), _f32)),
    mesh=plsc.VectorSubcoreMesh(core_axis_name="c", subcore_axis_name="s"),
    compiler_params=_sc_params,
    scratch_types=[
        pltpu.VMEM((1, _CH), jnp.int32),
        pltpu.VMEM((1, _CH), jnp.int32),
        pltpu.VMEM((1, _CH), jnp.int32),
        pltpu.VMEM((1, _CH), jnp.int32),
        pltpu.VMEM((1, _CH), _f32),
        pltpu.VMEM((1, _CH), _f32),
        pltpu.VMEM((1, _CH), jnp.int32),
        pltpu.VMEM((1, _CH), jnp.int32),
        pltpu.VMEM((_CH, _NF), _f32),
        pltpu.VMEM((_CH, _NF), _f32),
        pltpu.VMEM_SHARED((_N, _NF), _f32),
        pltpu.SemaphoreType.DMA,
        pltpu.SemaphoreType.DMA,
        pltpu.SemaphoreType.DMA,
        pltpu.SemaphoreType.DMA,
        pltpu.SemaphoreType.DMA,
        pltpu.SemaphoreType.DMA,
    ],
)(_agg_body)


_NB = 10
_BR = _N // _NB


def _stats_body(hlo_ref, hhi_ref, w_ref, gamma_ref, beta_ref, ab_ref,
                c_acc, cs_acc):
    i = pl.program_id(0)

    @pl.when(i == 0)
    def _():
        c_acc[...] = jnp.zeros_like(c_acc)
        cs_acc[...] = jnp.zeros_like(cs_acc)

    hlo = hlo_ref[...]
    hhi = hhi_ref[...]
    dn = (((0,), (0,)), ((), ()))
    c_acc[0:_NF, 0:_NF] += lax.dot_general(hlo, hlo, dn,
                                           preferred_element_type=_f32)
    c_acc[0:_NF, _NF:_DIN] += lax.dot_general(hlo, hhi, dn,
                                              preferred_element_type=_f32)
    c_acc[_NF:_DIN, 0:_NF] += lax.dot_general(hhi, hlo, dn,
                                              preferred_element_type=_f32)
    c_acc[_NF:_DIN, _NF:_DIN] += lax.dot_general(hhi, hhi, dn,
                                                 preferred_element_type=_f32)
    cs_acc[0:1, 0:_NF] += jnp.sum(hlo, axis=0, keepdims=True)
    cs_acc[0:1, _NF:_DIN] += jnp.sum(hhi, axis=0, keepdims=True)

    @pl.when(i == _NB - 1)
    def _():
        w = w_ref[...]
        inv_n = _f32(1.0 / _N)
        mean = jnp.dot(cs_acc[...], w, preferred_element_type=_f32) * inv_n
        cw = jnp.dot(c_acc[...], w, preferred_element_type=_f32)
        ey2 = jnp.sum(w * cw, axis=0, keepdims=True) * inv_n
        var = ey2 - mean * mean
        rstd = lax.rsqrt(var + _EPS)
        a = gamma_ref[...] * rstd
        b = beta_ref[...] - mean * a
        ab_ref[0:1, :] = a
        ab_ref[1:2, :] = b


def _fin_body(hlo_ref, hhi_ref, x_ref, w_ref, ws_ref, ab_ref, o_ref):
    y = (jnp.dot(hlo_ref[...], w_ref[0:_NF, :], preferred_element_type=_f32)
         + jnp.dot(hhi_ref[...], w_ref[_NF:_DIN, :],
                   preferred_element_type=_f32))
    s = jnp.dot(x_ref[...], ws_ref[...], preferred_element_type=_f32)
    a = ab_ref[0:1, :]
    b = ab_ref[1:2, :]
    o_ref[...] = jnp.maximum(y * a + b + s, 0.0)


def kernel(adj_indices, adj_values, node_embs, GCN_weight, skip_weight,
           bn_gamma, bn_beta):
    x = node_embs.astype(_f32)
    pad = _EPAD - _E
    row3 = jnp.concatenate(
        [adj_indices[0].astype(jnp.int32), jnp.zeros((pad,), jnp.int32)]
    ).reshape(_NS * _NCHUNK, 1, _CH)
    col3 = jnp.concatenate(
        [adj_indices[1].astype(jnp.int32), jnp.zeros((pad,), jnp.int32)]
    ).reshape(_NS * _NCHUNK, 1, _CH)
    val3 = jnp.concatenate(
        [adj_values.astype(_f32), jnp.zeros((pad,), _f32)]
    ).reshape(_NS * _NCHUNK, 1, _CH)
    x_lo = x[:, :_NF]
    x_hi = x[:, _NF:]

    h_lo, h_hi = _agg(x_lo, x_hi, row3, col3, val3)

    ab = pl.pallas_call(
        _stats_body,
        grid=(_NB,),
        in_specs=[
            pl.BlockSpec((_BR, _NF), lambda i: (i, 0)),
            pl.BlockSpec((_BR, _NF), lambda i: (i, 0)),
            pl.BlockSpec((_DIN, _DOUT), lambda i: (0, 0)),
            pl.BlockSpec((1, _DOUT), lambda i: (0, 0)),
            pl.BlockSpec((1, _DOUT), lambda i: (0, 0)),
        ],
        out_specs=pl.BlockSpec((2, _DOUT), lambda i: (0, 0)),
        out_shape=jax.ShapeDtypeStruct((2, _DOUT), _f32),
        scratch_shapes=[
            pltpu.VMEM((_DIN, _DIN), _f32),
            pltpu.VMEM((1, _DIN), _f32),
        ],
    )(h_lo, h_hi, GCN_weight, bn_gamma.reshape(1, _DOUT),
      bn_beta.reshape(1, _DOUT))

    out = pl.pallas_call(
        _fin_body,
        grid=(_NB,),
        in_specs=[
            pl.BlockSpec((_BR, _NF), lambda i: (i, 0)),
            pl.BlockSpec((_BR, _NF), lambda i: (i, 0)),
            pl.BlockSpec((_BR, _DIN), lambda i: (i, 0)),
            pl.BlockSpec((_DIN, _DOUT), lambda i: (0, 0)),
            pl.BlockSpec((_DIN, _DOUT), lambda i: (0, 0)),
            pl.BlockSpec((2, _DOUT), lambda i: (0, 0)),
        ],
        out_specs=pl.BlockSpec((_BR, _DOUT), lambda i: (i, 0)),
        out_shape=jax.ShapeDtypeStruct((_N, _DOUT), _f32),
    )(h_lo, h_hi, x, GCN_weight, skip_weight, ab)

    return out
